# baseline (speedup 1.0000x reference)
import functools

import jax
import jax.numpy as jnp
from jax import lax
from jax.experimental import pallas as pl
from jax.experimental.pallas import tpu as pltpu
from jax.experimental.pallas import tpu_sc as plsc

N_NODES = 10000
N_EDGES = 320000
D_FEAT = 128
D_EDGE = 16

NC = 2
NS = 16
NW = NC * NS
CHUNK = 128
N_CHUNKS = N_EDGES // CHUNK
N_CHUNKS_PAD = 2560
MAXC = N_CHUNKS_PAD // NW
N_ACC = 10240
ROWS_PER_TILE = N_ACC // NS

_mesh = plsc.VectorSubcoreMesh(core_axis_name="c", subcore_axis_name="s",
                               num_cores=NC, num_subcores=NS)


_SC_SCRATCH = [
    pltpu.VMEM((2, CHUNK), jnp.int32),
    pltpu.VMEM((CHUNK,), jnp.int32),
    pltpu.VMEM((CHUNK, D_FEAT), jnp.float32),
    pltpu.VMEM_SHARED((N_ACC, D_FEAT), jnp.float32),
    pltpu.SemaphoreType.DMA,
]


def _sc_body(data_hbm, sr_hbm, out_n, sridx, ramp, buf, acc_n, sem, gather):
    c = lax.axis_index("c")
    s = lax.axis_index("s")
    wid = s * NC + c
    r0 = s * ROWS_PER_TILE
    n_hop = ROWS_PER_TILE // CHUNK
    iota16 = lax.iota(jnp.int32, 16)
    zf16 = jnp.zeros((16,), jnp.float32)

    def _fill_ramp(base):
        for m in range(CHUNK // 16):
            ramp[pl.ds(m * 16, 16)] = iota16 + (base + m * 16)

    def _zero_rows(i, _):
        for m in range(D_FEAT // 16):
            buf[i, pl.ds(m * 16, 16)] = zf16
        return _
    lax.fori_loop(0, CHUNK, _zero_rows, None)

    for k in range(n_hop):
        _fill_ramp(r0 + k * CHUNK)
        pltpu.sync_copy(buf.at[pl.ds(0, CHUNK)], acc_n.at[ramp])
    plsc.subcore_barrier()

    lo = wid * N_CHUNKS // NW
    hi = (wid + 1) * N_CHUNKS // NW

    def step(j, _):
        pltpu.sync_copy(sr_hbm.at[j], sridx)
        if gather:
            pltpu.async_copy(data_hbm.at[sridx.at[0]],
                             buf.at[pl.ds(0, CHUNK)], sem).wait()
        else:
            pltpu.async_copy(data_hbm.at[pl.ds(j * CHUNK, CHUNK)],
                             buf.at[pl.ds(0, CHUNK)], sem).wait()
        pltpu.sync_copy(buf.at[pl.ds(0, CHUNK)], acc_n.at[sridx.at[1]],
                        add=True)
        return _

    lax.fori_loop(lo, hi, step, None)
    plsc.subcore_barrier()

    ro = c * N_ACC + r0
    for k in range(n_hop):
        _fill_ramp(r0 + k * CHUNK)
        pltpu.sync_copy(acc_n.at[ramp], buf.at[pl.ds(0, CHUNK)])
        pltpu.sync_copy(buf.at[pl.ds(0, CHUNK)], out_n.at[pl.ds(ro + k * CHUNK, CHUNK)])


_sc_nodes = functools.partial(
    pl.kernel, mesh=_mesh,
    out_type=jax.ShapeDtypeStruct((NC * N_ACC, D_FEAT), jnp.float32),
    scratch_types=_SC_SCRATCH,
)(functools.partial(_sc_body, gather=True))

_sc_eproj = functools.partial(
    pl.kernel, mesh=_mesh,
    out_type=jax.ShapeDtypeStruct((NC * N_ACC, D_FEAT), jnp.float32),
    scratch_types=_SC_SCRATCH,
)(functools.partial(_sc_body, gather=False))


_EP_R = 4000


def _eproj_body(edges_ref, We_ref, out_ref):
    out_ref[...] = jnp.dot(edges_ref[...], We_ref[...],
                           preferred_element_type=jnp.float32)


def _eproj(edges, We):
    return pl.pallas_call(
        _eproj_body,
        grid=(N_EDGES // _EP_R,),
        in_specs=[
            pl.BlockSpec((_EP_R, D_EDGE), lambda i: (i, 0)),
            pl.BlockSpec((D_EDGE, D_FEAT), lambda i: (0, 0)),
        ],
        out_specs=pl.BlockSpec((_EP_R, D_FEAT), lambda i: (i, 0)),
        out_shape=jax.ShapeDtypeStruct((N_EDGES, D_FEAT), jnp.float32),
    )(edges, We)


_R = 1000


def _mlp_body(pn_ref, pe_ref, nodes_ref, W1_ref, b1_ref, W2_ref, b2_ref,
              out_ref):
    h = pn_ref[0] + pn_ref[1] + pe_ref[0] + pe_ref[1] + nodes_ref[...]
    a = jnp.maximum(jnp.dot(h, W1_ref[...], preferred_element_type=jnp.float32)
                    + b1_ref[...], 0.0)
    out_ref[...] = (jnp.dot(a, W2_ref[...], preferred_element_type=jnp.float32)
                    + b2_ref[...])


def _mlp(pn, pe, nodes, W1, b1, W2, b2):
    return pl.pallas_call(
        _mlp_body,
        grid=(N_NODES // _R,),
        in_specs=[
            pl.BlockSpec((NC, _R, D_FEAT), lambda i: (0, i, 0)),
            pl.BlockSpec((NC, _R, D_FEAT), lambda i: (0, i, 0)),
            pl.BlockSpec((_R, D_FEAT), lambda i: (i, 0)),
            pl.BlockSpec((D_FEAT, D_FEAT), lambda i: (0, 0)),
            pl.BlockSpec((1, D_FEAT), lambda i: (0, 0)),
            pl.BlockSpec((D_FEAT, D_FEAT), lambda i: (0, 0)),
            pl.BlockSpec((1, D_FEAT), lambda i: (0, 0)),
        ],
        out_specs=pl.BlockSpec((_R, D_FEAT), lambda i: (i, 0)),
        out_shape=jax.ShapeDtypeStruct((N_NODES, D_FEAT), jnp.float32),
    )(pn, pe, nodes, W1, b1, W2, b2)


def kernel(nodes, senders, receivers, edges, We, W1, b1, W2, b2):
    sr = jnp.stack([senders.astype(jnp.int32).reshape(N_CHUNKS, CHUNK),
                    receivers.astype(jnp.int32).reshape(N_CHUNKS, CHUNK)],
                   axis=1)
    pn = _sc_nodes(nodes, sr)
    eproj = _eproj(edges, We)
    pe = _sc_eproj(eproj, sr)
    pn = pn.reshape(NC, N_ACC, D_FEAT)
    pe = pe.reshape(NC, N_ACC, D_FEAT)
    return _mlp(pn, pe, nodes, W1, b1.reshape(1, D_FEAT), W2,
                b2.reshape(1, D_FEAT))


# speedup vs baseline: 1.2000x; 1.2000x over previous
import functools

import jax
import jax.numpy as jnp
from jax import lax
from jax.experimental import pallas as pl
from jax.experimental.pallas import tpu as pltpu
from jax.experimental.pallas import tpu_sc as plsc

N_NODES = 10000
N_EDGES = 320000
D_FEAT = 128
D_EDGE = 16

NC = 2
NS = 16
NW = NC * NS
CHUNK = 128
N_CHUNKS = N_EDGES // CHUNK
N_CHUNKS_PAD = 2560
MAXC = N_CHUNKS_PAD // NW
N_ACC = 10240
ROWS_PER_TILE = N_ACC // NS

_mesh = plsc.VectorSubcoreMesh(core_axis_name="c", subcore_axis_name="s",
                               num_cores=NC, num_subcores=NS)


@functools.partial(
    pl.kernel,
    mesh=_mesh,
    out_type=jax.ShapeDtypeStruct((NC * N_ACC, D_FEAT), jnp.float32),
    scratch_types=[
        pltpu.VMEM((2, CHUNK), jnp.int32),
        pltpu.VMEM((CHUNK,), jnp.int32),
        pltpu.VMEM((2 * CHUNK, D_FEAT), jnp.float32),
        pltpu.VMEM_SHARED((N_ACC, D_FEAT), jnp.float32),
        pltpu.SemaphoreType.DMA,
    ],
)
def _sc_aggregate(nodes_hbm, sr_hbm, eproj_hbm,
                  out_n, sridx, ramp, buf, acc_n, sem):
    c = lax.axis_index("c")
    s = lax.axis_index("s")
    wid = s * NC + c
    r0 = s * ROWS_PER_TILE
    n_hop = ROWS_PER_TILE // CHUNK
    iota16 = lax.iota(jnp.int32, 16)
    zf16 = jnp.zeros((16,), jnp.float32)

    def _fill_ramp(base):
        for m in range(CHUNK // 16):
            ramp[pl.ds(m * 16, 16)] = iota16 + (base + m * 16)

    def _zero_rows(i, _):
        for m in range(D_FEAT // 16):
            buf[i, pl.ds(m * 16, 16)] = zf16
        return _
    lax.fori_loop(0, CHUNK, _zero_rows, None)

    for k in range(n_hop):
        _fill_ramp(r0 + k * CHUNK)
        pltpu.sync_copy(buf.at[pl.ds(0, CHUNK)], acc_n.at[ramp])
    plsc.subcore_barrier()

    lo = wid * N_CHUNKS // NW
    hi = (wid + 1) * N_CHUNKS // NW

    def step(j, _):
        pltpu.sync_copy(sr_hbm.at[j], sridx)
        g = pltpu.async_copy(nodes_hbm.at[sridx.at[0]],
                             buf.at[pl.ds(0, CHUNK)], sem)
        e = pltpu.async_copy(eproj_hbm.at[pl.ds(j * CHUNK, CHUNK)],
                             buf.at[pl.ds(CHUNK, CHUNK)], sem)
        g.wait()
        e.wait()
        pltpu.sync_copy(buf.at[pl.ds(0, CHUNK)], acc_n.at[sridx.at[1]],
                        add=True)
        pltpu.sync_copy(buf.at[pl.ds(CHUNK, CHUNK)], acc_n.at[sridx.at[1]],
                        add=True)
        return _

    lax.fori_loop(lo, hi, step, None)
    plsc.subcore_barrier()

    ro = c * N_ACC + r0
    for k in range(n_hop):
        _fill_ramp(r0 + k * CHUNK)
        pltpu.sync_copy(acc_n.at[ramp], buf.at[pl.ds(0, CHUNK)])
        pltpu.sync_copy(buf.at[pl.ds(0, CHUNK)], out_n.at[pl.ds(ro + k * CHUNK, CHUNK)])


_EP_R = 4000


def _eproj_body(edges_ref, We_ref, out_ref):
    out_ref[...] = jnp.dot(edges_ref[...], We_ref[...],
                           preferred_element_type=jnp.float32)


def _eproj(edges, We):
    return pl.pallas_call(
        _eproj_body,
        grid=(N_EDGES // _EP_R,),
        in_specs=[
            pl.BlockSpec((_EP_R, D_EDGE), lambda i: (i, 0)),
            pl.BlockSpec((D_EDGE, D_FEAT), lambda i: (0, 0)),
        ],
        out_specs=pl.BlockSpec((_EP_R, D_FEAT), lambda i: (i, 0)),
        out_shape=jax.ShapeDtypeStruct((N_EDGES, D_FEAT), jnp.float32),
    )(edges, We)


_R = 1000


def _mlp_body(pn_ref, nodes_ref, W1_ref, b1_ref, W2_ref, b2_ref, out_ref):
    h = pn_ref[0] + pn_ref[1] + nodes_ref[...]
    a = jnp.maximum(jnp.dot(h, W1_ref[...], preferred_element_type=jnp.float32)
                    + b1_ref[...], 0.0)
    out_ref[...] = (jnp.dot(a, W2_ref[...], preferred_element_type=jnp.float32)
                    + b2_ref[...])


def _mlp(pn, nodes, W1, b1, W2, b2):
    return pl.pallas_call(
        _mlp_body,
        grid=(N_NODES // _R,),
        in_specs=[
            pl.BlockSpec((NC, _R, D_FEAT), lambda i: (0, i, 0)),
            pl.BlockSpec((_R, D_FEAT), lambda i: (i, 0)),
            pl.BlockSpec((D_FEAT, D_FEAT), lambda i: (0, 0)),
            pl.BlockSpec((1, D_FEAT), lambda i: (0, 0)),
            pl.BlockSpec((D_FEAT, D_FEAT), lambda i: (0, 0)),
            pl.BlockSpec((1, D_FEAT), lambda i: (0, 0)),
        ],
        out_specs=pl.BlockSpec((_R, D_FEAT), lambda i: (i, 0)),
        out_shape=jax.ShapeDtypeStruct((N_NODES, D_FEAT), jnp.float32),
    )(pn, nodes, W1, b1, W2, b2)


def kernel(nodes, senders, receivers, edges, We, W1, b1, W2, b2):
    sr = jnp.stack([senders.astype(jnp.int32).reshape(N_CHUNKS, CHUNK),
                    receivers.astype(jnp.int32).reshape(N_CHUNKS, CHUNK)],
                   axis=1)
    eproj = _eproj(edges, We)
    pn = _sc_aggregate(nodes, sr, eproj)
    pn = pn.reshape(NC, N_ACC, D_FEAT)
    return _mlp(pn, nodes, W1, b1.reshape(1, D_FEAT), W2, b2.reshape(1, D_FEAT))
